# baseline (speedup 1.0000x reference)
import jax
import jax.numpy as jnp
from jax import lax
from jax.experimental import pallas as pl
from jax.experimental.pallas import tpu as pltpu
from jax.experimental.pallas import tpu_sc as plsc

N = 10000
E = 320000
D = 128
CHUNK = 128
NCHUNKS = E // CHUNK
NCORES = 2
NSUB = 16
NW = NCORES * NSUB
NP = 10112
RPT = NP // NSUB
EB = 8000
NB = 2000
ITERS = (NCHUNKS + NW - 1) // NW
ROW_CHUNKS = []
_o = 0
while _o < RPT:
    ROW_CHUNKS.append((_o, min(CHUNK, RPT - _o)))
    _o += ROW_CHUNKS[-1][1]


def _gather_u_body(src_hbm, x_hbm, hu_hbm,
                   idx0, idx1, r0, r1, s0, s1):
    core = lax.axis_index("c")
    sub = lax.axis_index("s")
    wid = core * NSUB + sub
    n = (NCHUNKS - wid + NW - 1) // NW
    bufs = ((idx0, r0, s0), (idx1, r1, s1))

    def start(i, p):
        idx, r, sem = bufs[p]

        @pl.when(i < n)
        def _():
            base = (wid + NW * i) * CHUNK
            pltpu.sync_copy(src_hbm.at[pl.ds(base, CHUNK)], idx)
            pltpu.async_copy(x_hbm.at[idx], r, sem)

    def finish(i, p):
        idx, r, sem = bufs[p]

        @pl.when(i < n)
        def _():
            base = (wid + NW * i) * CHUNK
            pltpu.make_async_copy(x_hbm.at[pl.ds(0, CHUNK)], r, sem).wait()
            pltpu.sync_copy(r, hu_hbm.at[pl.ds(base, CHUNK)])

    start(0, 0)

    def body(k, carry):
        i0 = 2 * k
        start(i0 + 1, 1)
        finish(i0, 0)
        start(i0 + 2, 0)
        finish(i0 + 1, 1)
        return carry

    lax.fori_loop(0, (ITERS + 1) // 2, body, 0)


def _gather_v_body(dst_hbm, x_hbm, hv_hbm, degp_hbm,
                   idx0, idx1, r0, r1, ones_b, deg_sh, s0, s1):
    core = lax.axis_index("c")
    sub = lax.axis_index("s")
    wid = core * NSUB + sub
    n = (NCHUNKS - wid + NW - 1) // NW

    zv = jnp.zeros((16,), jnp.float32)
    ov = jnp.ones((16,), jnp.float32)

    def fill_row(r, carry):
        for j in range(D // 16):
            r0[r, pl.ds(j * 16, 16)] = zv
            ones_b[r, pl.ds(j * 16, 16)] = ov
        return carry

    lax.fori_loop(0, CHUNK, fill_row, 0)
    for o, nn in ROW_CHUNKS:
        pltpu.sync_copy(r0.at[pl.ds(0, nn)],
                        deg_sh.at[pl.ds(sub * RPT + o, nn)])
    plsc.subcore_barrier()

    bufs = ((idx0, r0, s0), (idx1, r1, s1))

    def start(i, p):
        idx, r, sem = bufs[p]

        @pl.when(i < n)
        def _():
            base = (wid + NW * i) * CHUNK
            pltpu.sync_copy(dst_hbm.at[pl.ds(base, CHUNK)], idx)
            pltpu.async_copy(x_hbm.at[idx], r, sem)
            pltpu.sync_copy(ones_b, deg_sh.at[idx], add=True)

    def finish(i, p):
        idx, r, sem = bufs[p]

        @pl.when(i < n)
        def _():
            base = (wid + NW * i) * CHUNK
            pltpu.make_async_copy(x_hbm.at[pl.ds(0, CHUNK)], r, sem).wait()
            pltpu.sync_copy(r, hv_hbm.at[pl.ds(base, CHUNK)])

    start(0, 0)

    def body(k, carry):
        i0 = 2 * k
        start(i0 + 1, 1)
        finish(i0, 0)
        start(i0 + 2, 0)
        finish(i0 + 1, 1)
        return carry

    lax.fori_loop(0, (ITERS + 1) // 2, body, 0)
    plsc.subcore_barrier()

    for o, nn in ROW_CHUNKS:
        off = sub * RPT + o
        pltpu.sync_copy(deg_sh.at[pl.ds(off, nn)], r0.at[pl.ds(0, nn)])
        pltpu.sync_copy(r0.at[pl.ds(0, nn)],
                        degp_hbm.at[pl.ds(core * NP + off, nn)])


def _scatter_body(dst_hbm, m_hbm, part_hbm,
                  idx0, idx1, rows0, rows1, acc_sh, sm0, sm1):
    core = lax.axis_index("c")
    sub = lax.axis_index("s")
    wid = core * NSUB + sub
    n = (NCHUNKS - wid + NW - 1) // NW

    zv = jnp.zeros((16,), jnp.float32)

    def zero_row(r, carry):
        for j in range(D // 16):
            rows0[r, pl.ds(j * 16, 16)] = zv
        return carry

    lax.fori_loop(0, CHUNK, zero_row, 0)
    for o, nn in ROW_CHUNKS:
        pltpu.sync_copy(rows0.at[pl.ds(0, nn)],
                        acc_sh.at[pl.ds(sub * RPT + o, nn)])
    plsc.subcore_barrier()

    bufs = ((idx0, rows0, sm0), (idx1, rows1, sm1))

    def start(i, p):
        idx_b, rows_b, sm = bufs[p]
        cid = wid + NW * i

        @pl.when(i < n)
        def _():
            base = cid * CHUNK
            pltpu.sync_copy(dst_hbm.at[pl.ds(base, CHUNK)], idx_b)
            pltpu.async_copy(m_hbm.at[pl.ds(base, CHUNK)], rows_b, sm)

    def finish(i, p):
        idx_b, rows_b, sm = bufs[p]

        @pl.when(i < n)
        def _():
            pltpu.make_async_copy(m_hbm.at[pl.ds(0, CHUNK)], rows_b, sm).wait()
            pltpu.sync_copy(rows_b, acc_sh.at[idx_b], add=True)

    start(0, 0)

    def body(k, carry):
        i0 = 2 * k
        start(i0 + 1, 1)
        finish(i0, 0)
        start(i0 + 2, 0)
        finish(i0 + 1, 1)
        return carry

    lax.fori_loop(0, (ITERS + 1) // 2, body, 0)
    plsc.subcore_barrier()

    for o, nn in ROW_CHUNKS:
        off = sub * RPT + o
        pltpu.sync_copy(acc_sh.at[pl.ds(off, nn)], rows0.at[pl.ds(0, nn)])
        pltpu.sync_copy(rows0.at[pl.ds(0, nn)],
                        part_hbm.at[pl.ds(core * NP + off, nn)])


_SC = {}


def _sc_kernels():
    if not _SC:
        mesh = plsc.VectorSubcoreMesh(
            core_axis_name="c", subcore_axis_name="s",
            num_cores=NCORES, num_subcores=NSUB)

        _SC["gather_u"] = pl.kernel(
            _gather_u_body,
            out_type=jax.ShapeDtypeStruct((E, D), jnp.float32),
            mesh=mesh,
            scratch_types=[
                pltpu.VMEM((CHUNK,), jnp.int32),
                pltpu.VMEM((CHUNK,), jnp.int32),
                pltpu.VMEM((CHUNK, D), jnp.float32),
                pltpu.VMEM((CHUNK, D), jnp.float32),
                pltpu.SemaphoreType.DMA,
                pltpu.SemaphoreType.DMA,
            ],
        )
        _SC["gather_v"] = pl.kernel(
            _gather_v_body,
            out_type=[jax.ShapeDtypeStruct((E, D), jnp.float32),
                      jax.ShapeDtypeStruct((NCORES * NP, D), jnp.float32)],
            mesh=mesh,
            scratch_types=[
                pltpu.VMEM((CHUNK,), jnp.int32),
                pltpu.VMEM((CHUNK,), jnp.int32),
                pltpu.VMEM((CHUNK, D), jnp.float32),
                pltpu.VMEM((CHUNK, D), jnp.float32),
                pltpu.VMEM((CHUNK, D), jnp.float32),
                pltpu.VMEM_SHARED((NP, D), jnp.float32),
                pltpu.SemaphoreType.DMA,
                pltpu.SemaphoreType.DMA,
            ],
        )
        _SC["scatter"] = pl.kernel(
            _scatter_body,
            out_type=jax.ShapeDtypeStruct((NCORES * NP, D), jnp.float32),
            mesh=mesh,
            scratch_types=[
                pltpu.VMEM((CHUNK,), jnp.int32),
                pltpu.VMEM((CHUNK,), jnp.int32),
                pltpu.VMEM((CHUNK, D), jnp.float32),
                pltpu.VMEM((CHUNK, D), jnp.float32),
                pltpu.VMEM_SHARED((NP, D), jnp.float32),
                pltpu.SemaphoreType.DMA,
                pltpu.SemaphoreType.DMA,
            ],
        )
    return _SC["gather_u"], _SC["gather_v"], _SC["scatter"]


def _mlp_body(hu_ref, hv_ref, w1, b1, w2, b2, w3, b3, w4, b4, e_ref, m_ref):
    f32 = jnp.float32
    bf16 = jnp.bfloat16
    hu = hu_ref[...]
    d = (hu - hv_ref[...]).astype(bf16)
    h = jnp.dot(d, w1[...], preferred_element_type=f32).astype(bf16) + b1[...]
    h = jnp.maximum(h, jnp.bfloat16(0.01) * h)
    h = jnp.dot(h, w2[...], preferred_element_type=f32).astype(bf16) + b2[...]
    h = jnp.maximum(h, jnp.bfloat16(0.01) * h)
    h = jnp.dot(h, w3[...], preferred_element_type=f32) + b3[...]
    h = jnp.maximum(h, 0.01 * h)
    z = jnp.sum(h * w4[...], axis=1, keepdims=True) + b4[...]
    e = 1.0 / (1.0 + jnp.exp(-z))
    e_ref[...] = e
    m_ref[...] = e * hu


def _mlp(hu, hv, w1t, b1, w2t, b2, w3t, b3, w4, b4):
    return pl.pallas_call(
        _mlp_body,
        grid=(E // EB,),
        in_specs=[
            pl.BlockSpec((EB, D), lambda i: (i, 0)),
            pl.BlockSpec((EB, D), lambda i: (i, 0)),
            pl.BlockSpec(w1t.shape, lambda i: (0, 0)),
            pl.BlockSpec(b1.shape, lambda i: (0, 0)),
            pl.BlockSpec(w2t.shape, lambda i: (0, 0)),
            pl.BlockSpec(b2.shape, lambda i: (0, 0)),
            pl.BlockSpec(w3t.shape, lambda i: (0, 0)),
            pl.BlockSpec(b3.shape, lambda i: (0, 0)),
            pl.BlockSpec(w4.shape, lambda i: (0, 0)),
            pl.BlockSpec(b4.shape, lambda i: (0, 0)),
        ],
        out_specs=[
            pl.BlockSpec((EB, 1), lambda i: (i, 0)),
            pl.BlockSpec((EB, D), lambda i: (i, 0)),
        ],
        out_shape=[
            jax.ShapeDtypeStruct((E, 1), jnp.float32),
            jax.ShapeDtypeStruct((E, D), jnp.float32),
        ],
    )(hu, hv, w1t, b1, w2t, b2, w3t, b3, w4, b4)


def _final_body(x_ref, p_ref, d_ref, wst, wnt, b, out_ref):
    p = p_ref[...]
    dg = d_ref[...]
    sacc = p[0] + p[1]
    deg = dg[0] + dg[1]
    hn = sacc / jnp.maximum(deg, 1.0)
    rst = x_ref[...] @ wst[...] + hn @ wnt[...] + b[...]
    a = jnp.where(rst >= 0, rst, 0.01 * rst)
    nrm = jnp.sqrt(jnp.sum(a * a, axis=1, keepdims=True))
    out_ref[...] = a / jnp.maximum(nrm, 1e-12)


def _final(x, parts, degp, wst, wnt, b):
    return pl.pallas_call(
        _final_body,
        grid=(N // NB,),
        in_specs=[
            pl.BlockSpec((NB, D), lambda i: (i, 0)),
            pl.BlockSpec((2, NB, D), lambda i: (0, i, 0)),
            pl.BlockSpec((2, NB, D), lambda i: (0, i, 0)),
            pl.BlockSpec(wst.shape, lambda i: (0, 0)),
            pl.BlockSpec(wnt.shape, lambda i: (0, 0)),
            pl.BlockSpec(b.shape, lambda i: (0, 0)),
        ],
        out_specs=pl.BlockSpec((NB, D), lambda i: (i, 0)),
        out_shape=jax.ShapeDtypeStruct((N, D), jnp.float32),
    )(x, parts, degp, wst, wnt, b)


def kernel(x, edge_index, W1, b1, W2, b2, W3, b3, W4, b4, Wself, Wneigh, bias):
    src = edge_index[0]
    dst = edge_index[1]
    gather_u, gather_v, scatter = _sc_kernels()
    hu = gather_u(src, x)
    hv, degp_flat = gather_v(dst, x)
    bf16 = jnp.bfloat16
    bf = lambda a: a.astype(jnp.bfloat16)
    e, m = _mlp(hu, hv, bf(W1.T), bf(b1[None, :]),
                bf(W2.T), bf(b2[None, :]),
                bf(W3.T), b3[None, :], W4, b4[None, :])
    part_flat = scatter(dst, m)
    parts = part_flat.reshape(NCORES, NP, D)
    degp = degp_flat.reshape(NCORES, NP, D)
    a = _final(x, parts, degp, Wself.T, Wneigh.T, bias[None, :])
    return (a, e)


# speedup vs baseline: 1.1261x; 1.1261x over previous
import jax
import jax.numpy as jnp
from jax import lax
from jax.experimental import pallas as pl
from jax.experimental.pallas import tpu as pltpu
from jax.experimental.pallas import tpu_sc as plsc

N = 10000
E = 320000
D = 128
CHUNK = 128
NCHUNKS = E // CHUNK
NSLICES = 2
EH = E // NSLICES
NCHUNKS_H = EH // CHUNK
NCORES = 2
NSUB = 16
NW = NCORES * NSUB
NP = 10112
RPT = NP // NSUB
EB = 8000
NB = 2000
ITERS = (NCHUNKS_H + NW - 1) // NW
ROW_CHUNKS = []
_o = 0
while _o < RPT:
    ROW_CHUNKS.append((_o, min(CHUNK, RPT - _o)))
    _o += ROW_CHUNKS[-1][1]


def _gather_u_body(src_hbm, x_hbm, hu_hbm,
                   idx0, idx1, r0, r1, s0, s1):
    core = lax.axis_index("c")
    sub = lax.axis_index("s")
    wid = core * NSUB + sub
    n = (NCHUNKS_H - wid + NW - 1) // NW
    bufs = ((idx0, r0, s0), (idx1, r1, s1))

    def start(i, p):
        idx, r, sem = bufs[p]

        @pl.when(i < n)
        def _():
            base = (wid + NW * i) * CHUNK
            pltpu.sync_copy(src_hbm.at[pl.ds(base, CHUNK)], idx)
            pltpu.async_copy(x_hbm.at[idx], r, sem)

    def finish(i, p):
        idx, r, sem = bufs[p]

        @pl.when(i < n)
        def _():
            base = (wid + NW * i) * CHUNK
            pltpu.make_async_copy(x_hbm.at[pl.ds(0, CHUNK)], r, sem).wait()
            pltpu.sync_copy(r, hu_hbm.at[pl.ds(base, CHUNK)])

    start(0, 0)

    def body(k, carry):
        i0 = 2 * k
        start(i0 + 1, 1)
        finish(i0, 0)
        start(i0 + 2, 0)
        finish(i0 + 1, 1)
        return carry

    lax.fori_loop(0, (ITERS + 1) // 2, body, 0)


def _gather_v_body(dst_hbm, x_hbm, hv_hbm, degp_hbm,
                   idx0, idx1, r0, r1, ones_b, deg_sh, s0, s1):
    core = lax.axis_index("c")
    sub = lax.axis_index("s")
    wid = core * NSUB + sub
    n = (NCHUNKS_H - wid + NW - 1) // NW

    zv = jnp.zeros((16,), jnp.float32)
    ov = jnp.ones((16,), jnp.float32)

    def fill_row(r, carry):
        for j in range(D // 16):
            r0[r, pl.ds(j * 16, 16)] = zv
            ones_b[r, pl.ds(j * 16, 16)] = ov
        return carry

    lax.fori_loop(0, CHUNK, fill_row, 0)
    for o, nn in ROW_CHUNKS:
        pltpu.sync_copy(r0.at[pl.ds(0, nn)],
                        deg_sh.at[pl.ds(sub * RPT + o, nn)])
    plsc.subcore_barrier()

    bufs = ((idx0, r0, s0), (idx1, r1, s1))

    def start(i, p):
        idx, r, sem = bufs[p]

        @pl.when(i < n)
        def _():
            base = (wid + NW * i) * CHUNK
            pltpu.sync_copy(dst_hbm.at[pl.ds(base, CHUNK)], idx)
            pltpu.async_copy(x_hbm.at[idx], r, sem)
            pltpu.sync_copy(ones_b, deg_sh.at[idx], add=True)

    def finish(i, p):
        idx, r, sem = bufs[p]

        @pl.when(i < n)
        def _():
            base = (wid + NW * i) * CHUNK
            pltpu.make_async_copy(x_hbm.at[pl.ds(0, CHUNK)], r, sem).wait()
            pltpu.sync_copy(r, hv_hbm.at[pl.ds(base, CHUNK)])

    start(0, 0)

    def body(k, carry):
        i0 = 2 * k
        start(i0 + 1, 1)
        finish(i0, 0)
        start(i0 + 2, 0)
        finish(i0 + 1, 1)
        return carry

    lax.fori_loop(0, (ITERS + 1) // 2, body, 0)
    plsc.subcore_barrier()

    for o, nn in ROW_CHUNKS:
        off = sub * RPT + o
        pltpu.sync_copy(deg_sh.at[pl.ds(off, nn)], r0.at[pl.ds(0, nn)])
        pltpu.sync_copy(r0.at[pl.ds(0, nn)],
                        degp_hbm.at[pl.ds(core * NP + off, nn)])


def _scatter_body(dst_hbm, m_hbm, part_hbm,
                  idx0, idx1, rows0, rows1, acc_sh, sm0, sm1):
    core = lax.axis_index("c")
    sub = lax.axis_index("s")
    wid = core * NSUB + sub
    n = (NCHUNKS_H - wid + NW - 1) // NW

    zv = jnp.zeros((16,), jnp.float32)

    def zero_row(r, carry):
        for j in range(D // 16):
            rows0[r, pl.ds(j * 16, 16)] = zv
        return carry

    lax.fori_loop(0, CHUNK, zero_row, 0)
    for o, nn in ROW_CHUNKS:
        pltpu.sync_copy(rows0.at[pl.ds(0, nn)],
                        acc_sh.at[pl.ds(sub * RPT + o, nn)])
    plsc.subcore_barrier()

    bufs = ((idx0, rows0, sm0), (idx1, rows1, sm1))

    def start(i, p):
        idx_b, rows_b, sm = bufs[p]
        cid = wid + NW * i

        @pl.when(i < n)
        def _():
            base = cid * CHUNK
            pltpu.sync_copy(dst_hbm.at[pl.ds(base, CHUNK)], idx_b)
            pltpu.async_copy(m_hbm.at[pl.ds(base, CHUNK)], rows_b, sm)

    def finish(i, p):
        idx_b, rows_b, sm = bufs[p]

        @pl.when(i < n)
        def _():
            pltpu.make_async_copy(m_hbm.at[pl.ds(0, CHUNK)], rows_b, sm).wait()
            pltpu.sync_copy(rows_b, acc_sh.at[idx_b], add=True)

    start(0, 0)

    def body(k, carry):
        i0 = 2 * k
        start(i0 + 1, 1)
        finish(i0, 0)
        start(i0 + 2, 0)
        finish(i0 + 1, 1)
        return carry

    lax.fori_loop(0, (ITERS + 1) // 2, body, 0)
    plsc.subcore_barrier()

    for o, nn in ROW_CHUNKS:
        off = sub * RPT + o
        pltpu.sync_copy(acc_sh.at[pl.ds(off, nn)], rows0.at[pl.ds(0, nn)])
        pltpu.sync_copy(rows0.at[pl.ds(0, nn)],
                        part_hbm.at[pl.ds(core * NP + off, nn)])


_SC = {}


def _sc_kernels():
    if not _SC:
        mesh = plsc.VectorSubcoreMesh(
            core_axis_name="c", subcore_axis_name="s",
            num_cores=NCORES, num_subcores=NSUB)

        _SC["gather_u"] = pl.kernel(
            _gather_u_body,
            out_type=jax.ShapeDtypeStruct((EH, D), jnp.float32),
            mesh=mesh,
            scratch_types=[
                pltpu.VMEM((CHUNK,), jnp.int32),
                pltpu.VMEM((CHUNK,), jnp.int32),
                pltpu.VMEM((CHUNK, D), jnp.float32),
                pltpu.VMEM((CHUNK, D), jnp.float32),
                pltpu.SemaphoreType.DMA,
                pltpu.SemaphoreType.DMA,
            ],
        )
        _SC["gather_v"] = pl.kernel(
            _gather_v_body,
            out_type=[jax.ShapeDtypeStruct((EH, D), jnp.float32),
                      jax.ShapeDtypeStruct((NCORES * NP, D), jnp.float32)],
            mesh=mesh,
            scratch_types=[
                pltpu.VMEM((CHUNK,), jnp.int32),
                pltpu.VMEM((CHUNK,), jnp.int32),
                pltpu.VMEM((CHUNK, D), jnp.float32),
                pltpu.VMEM((CHUNK, D), jnp.float32),
                pltpu.VMEM((CHUNK, D), jnp.float32),
                pltpu.VMEM_SHARED((NP, D), jnp.float32),
                pltpu.SemaphoreType.DMA,
                pltpu.SemaphoreType.DMA,
            ],
        )
        _SC["scatter"] = pl.kernel(
            _scatter_body,
            out_type=jax.ShapeDtypeStruct((NCORES * NP, D), jnp.float32),
            mesh=mesh,
            scratch_types=[
                pltpu.VMEM((CHUNK,), jnp.int32),
                pltpu.VMEM((CHUNK,), jnp.int32),
                pltpu.VMEM((CHUNK, D), jnp.float32),
                pltpu.VMEM((CHUNK, D), jnp.float32),
                pltpu.VMEM_SHARED((NP, D), jnp.float32),
                pltpu.SemaphoreType.DMA,
                pltpu.SemaphoreType.DMA,
            ],
        )
    return _SC["gather_u"], _SC["gather_v"], _SC["scatter"]


def _mlp_body(hu_ref, hv_ref, w1, b1, w2, b2, w3, b3, w4, b4, e_ref, m_ref):
    f32 = jnp.float32
    bf16 = jnp.bfloat16
    hu = hu_ref[...]
    d = (hu - hv_ref[...]).astype(bf16)
    h = jnp.dot(d, w1[...], preferred_element_type=f32).astype(bf16) + b1[...]
    h = jnp.maximum(h, jnp.bfloat16(0.01) * h)
    h = jnp.dot(h, w2[...], preferred_element_type=f32).astype(bf16) + b2[...]
    h = jnp.maximum(h, jnp.bfloat16(0.01) * h)
    h = jnp.dot(h, w3[...], preferred_element_type=f32) + b3[...]
    h = jnp.maximum(h, 0.01 * h)
    z = jnp.sum(h * w4[...], axis=1, keepdims=True) + b4[...]
    e = 1.0 / (1.0 + jnp.exp(-z))
    e_ref[...] = e
    m_ref[...] = e * hu


def _mlp(hu, hv, w1t, b1, w2t, b2, w3t, b3, w4, b4):
    return pl.pallas_call(
        _mlp_body,
        grid=(EH // EB,),
        in_specs=[
            pl.BlockSpec((EB, D), lambda i: (i, 0)),
            pl.BlockSpec((EB, D), lambda i: (i, 0)),
            pl.BlockSpec(w1t.shape, lambda i: (0, 0)),
            pl.BlockSpec(b1.shape, lambda i: (0, 0)),
            pl.BlockSpec(w2t.shape, lambda i: (0, 0)),
            pl.BlockSpec(b2.shape, lambda i: (0, 0)),
            pl.BlockSpec(w3t.shape, lambda i: (0, 0)),
            pl.BlockSpec(b3.shape, lambda i: (0, 0)),
            pl.BlockSpec(w4.shape, lambda i: (0, 0)),
            pl.BlockSpec(b4.shape, lambda i: (0, 0)),
        ],
        out_specs=[
            pl.BlockSpec((EB, 1), lambda i: (i, 0)),
            pl.BlockSpec((EB, D), lambda i: (i, 0)),
        ],
        out_shape=[
            jax.ShapeDtypeStruct((EH, 1), jnp.float32),
            jax.ShapeDtypeStruct((EH, D), jnp.float32),
        ],
    )(hu, hv, w1t, b1, w2t, b2, w3t, b3, w4, b4)


def _final_body(x_ref, p_ref, d_ref, wst, wnt, b, out_ref):
    p = p_ref[...]
    dg = d_ref[...]
    sacc = p[0] + p[1] + p[2] + p[3]
    deg = dg[0] + dg[1] + dg[2] + dg[3]
    hn = sacc / jnp.maximum(deg, 1.0)
    rst = x_ref[...] @ wst[...] + hn @ wnt[...] + b[...]
    a = jnp.where(rst >= 0, rst, 0.01 * rst)
    nrm = jnp.sqrt(jnp.sum(a * a, axis=1, keepdims=True))
    out_ref[...] = a / jnp.maximum(nrm, 1e-12)


def _final(x, parts, degp, wst, wnt, b):
    return pl.pallas_call(
        _final_body,
        grid=(N // NB,),
        in_specs=[
            pl.BlockSpec((NB, D), lambda i: (i, 0)),
            pl.BlockSpec((2 * NSLICES, NB, D), lambda i: (0, i, 0)),
            pl.BlockSpec((2 * NSLICES, NB, D), lambda i: (0, i, 0)),
            pl.BlockSpec(wst.shape, lambda i: (0, 0)),
            pl.BlockSpec(wnt.shape, lambda i: (0, 0)),
            pl.BlockSpec(b.shape, lambda i: (0, 0)),
        ],
        out_specs=pl.BlockSpec((NB, D), lambda i: (i, 0)),
        out_shape=jax.ShapeDtypeStruct((N, D), jnp.float32),
    )(x, parts, degp, wst, wnt, b)


def kernel(x, edge_index, W1, b1, W2, b2, W3, b3, W4, b4, Wself, Wneigh, bias):
    src = edge_index[0]
    dst = edge_index[1]
    gather_u, gather_v, scatter = _sc_kernels()
    bf = lambda a: a.astype(jnp.bfloat16)
    mlp_w = (bf(W1.T), bf(b1[None, :]), bf(W2.T), bf(b2[None, :]),
             bf(W3.T), b3[None, :], W4, b4[None, :])
    es, parts, degps = [], [], []
    for h in range(NSLICES):
        src_h = src[h * EH:(h + 1) * EH]
        dst_h = dst[h * EH:(h + 1) * EH]
        hu = gather_u(src_h, x)
        hv, degp_flat = gather_v(dst_h, x)
        e, m = _mlp(hu, hv, *mlp_w)
        part_flat = scatter(dst_h, m)
        es.append(e)
        parts.append(part_flat.reshape(NCORES, NP, D))
        degps.append(degp_flat.reshape(NCORES, NP, D))
    parts4 = jnp.concatenate(parts, axis=0)
    degp4 = jnp.concatenate(degps, axis=0)
    a = _final(x, parts4, degp4, Wself.T, Wneigh.T, bias[None, :])
    return (a, jnp.concatenate(es, axis=0))
